# baseline (speedup 1.0000x reference)
import jax
import jax.numpy as jnp
from jax import lax
from jax.experimental import pallas as pl
from jax.experimental.pallas import tpu as pltpu
from jax.experimental.pallas import tpu_sc as plsc

_N = 10000
_E = 320000
_D = 128
_KW = 64
_NC = 2
_NS = 16
_K = 256
_CHT = 80
_EPAD = _NS * _CHT * _K
_NPAD = 10240
_RPT = _NPAD // _NS
_NBUF = 4

_HI = lax.Precision.HIGHEST


def _make_propagate(with_cnt: bool):
  mesh = plsc.VectorSubcoreMesh(core_axis_name="c", subcore_axis_name="s")
  out_type = [jax.ShapeDtypeStruct((_NC, _NPAD, _KW), jnp.bfloat16)]
  if with_cnt:
    out_type.append(jax.ShapeDtypeStruct((_NC, _NPAD, 16), jnp.float32))
  scratch = [
      pltpu.VMEM((_CHT, _K), jnp.int32),
      pltpu.VMEM((_CHT, _K), jnp.int32),
  ]
  scratch += [pltpu.VMEM((_K, _KW), jnp.bfloat16) for _ in range(_NBUF)]
  if with_cnt:
    scratch.append(pltpu.VMEM((_K, 16), jnp.float32))
    scratch.append(pltpu.VMEM_SHARED((_NPAD, 16), jnp.float32))
  scratch.append(pltpu.VMEM_SHARED((_NPAD, _KW), jnp.bfloat16))
  scratch += [pltpu.SemaphoreType.DMA for _ in range(2 * _NBUF)]

  def body(y_hbm, src_hbm, dst_hbm, *refs):
    if with_cnt:
      z_out, cnt_out = refs[0], refs[1]
      rest = refs[2:]
    else:
      z_out = refs[0]
      rest = refs[1:]
    idx_s, idx_d = rest[0], rest[1]
    bufs = rest[2:2 + _NBUF]
    off = 2 + _NBUF
    if with_cnt:
      ones_b, cnt_sh = rest[off], rest[off + 1]
      off += 2
    z_sh = rest[off]
    sems = rest[off + 1:]
    sem_g, sem_s = sems[:_NBUF], sems[_NBUF:]

    cid = lax.axis_index("c")
    sid = lax.axis_index("s")
    base = sid * _RPT

    pltpu.sync_copy(src_hbm.at[cid, sid], idx_s)
    pltpu.sync_copy(dst_hbm.at[sid], idx_d)

    zv = jnp.zeros((16,), jnp.float32)
    zvh = jnp.zeros((32,), jnp.bfloat16)

    def _zrow(i, c):
      def _zcol(j, c2):
        bufs[0][i, pl.ds(j * 32, 32)] = zvh
        return c2
      return lax.fori_loop(0, _KW // 32, _zcol, c)
    lax.fori_loop(0, _K, _zrow, 0)

    nfull, rem = divmod(_RPT, _K)
    for r in range(nfull):
      pltpu.sync_copy(bufs[0], z_sh.at[pl.ds(base + r * _K, _K)])
    if rem:
      pltpu.sync_copy(bufs[0].at[pl.ds(0, rem)],
                      z_sh.at[pl.ds(base + nfull * _K, rem)])

    if with_cnt:
      ov = jnp.ones((16,), jnp.float32)

      def _zofill(i, c):
        ones_b[i, :] = zv
        return c
      lax.fori_loop(0, _K, _zofill, 0)
      for r in range(nfull):
        pltpu.sync_copy(ones_b, cnt_sh.at[pl.ds(base + r * _K, _K)])
      if rem:
        pltpu.sync_copy(ones_b.at[pl.ds(0, rem)],
                        cnt_sh.at[pl.ds(base + nfull * _K, rem)])

      def _onefill(i, c):
        ones_b[i, :] = ov
        return c
      lax.fori_loop(0, _K, _onefill, 0)

    plsc.subcore_barrier()

    def _cnt_pred(j):
      return (j >= cid * (_CHT // 2)) & (j < (cid + 1) * (_CHT // 2))

    def _cnt_chunk(j, p):
      @pl.when(_cnt_pred(j))
      def _():
        pltpu.async_copy(ones_b, cnt_sh.at[idx_d.at[j]], sem_s[p], add=True)

    def _wait_cnt_chunk(j, p):
      @pl.when(_cnt_pred(j))
      def _():
        pltpu.make_async_copy(ones_b, cnt_sh.at[idx_d.at[j]], sem_s[p]).wait()

    def _gather(j, p):
      pltpu.async_copy(y_hbm.at[idx_s.at[j]], bufs[p], sem_g[p])

    def _wait_gather(j, p):
      pltpu.make_async_copy(y_hbm.at[idx_s.at[j]], bufs[p], sem_g[p]).wait()

    def _scatter(j, p):
      pltpu.async_copy(bufs[p], z_sh.at[idx_d.at[j]], sem_s[p], add=True)

    def _wait_scatter(j, p):
      pltpu.make_async_copy(bufs[p], z_sh.at[idx_d.at[j]], sem_s[p]).wait()

    for p in range(_NBUF):
      _gather(p, p)

    def _step(t, c):
      for p in range(_NBUF):
        j = _NBUF * t + p
        _wait_gather(j, p)
        _scatter(j, p)
        if with_cnt:
          _cnt_chunk(j, p)
        _wait_scatter(j, p)
        if with_cnt:
          _wait_cnt_chunk(j, p)
        _gather(j + _NBUF, p)
      return c
    lax.fori_loop(0, _CHT // _NBUF - 1, _step, 0)

    for p in range(_NBUF):
      j = _CHT - _NBUF + p
      _wait_gather(j, p)
      _scatter(j, p)
      if with_cnt:
        _cnt_chunk(j, p)
      _wait_scatter(j, p)
      if with_cnt:
        _wait_cnt_chunk(j, p)

    plsc.subcore_barrier()
    pltpu.sync_copy(z_sh.at[pl.ds(base, _RPT)],
                    z_out.at[cid, pl.ds(base, _RPT)])
    if with_cnt:
      pltpu.sync_copy(cnt_sh.at[pl.ds(base, _RPT)],
                      cnt_out.at[cid, pl.ds(base, _RPT)])

  return pl.kernel(body, out_type=tuple(out_type), mesh=mesh,
                   scratch_types=tuple(scratch),
                   compiler_params=pltpu.CompilerParams(
                       use_tc_tiling_on_sc=False))


_prop_cnt = _make_propagate(True)
_prop = _make_propagate(False)


_BLK = 1000
_G = _N // _BLK


def _dot(a, b):
  return lax.dot_general(a, b, (((1,), (0,)), ((), ())), precision=_HI)


def _stage_a_body(x_ref, wl_ref, wr_ref, b_ref, y_ref, s_ref):
  xb = x_ref[...]
  y = _dot(xb, wl_ref[...]).astype(jnp.bfloat16)
  y_ref[0] = y[:, :_KW]
  y_ref[1] = y[:, _KW:]
  s_ref[...] = _dot(xb, wr_ref[...]) + b_ref[...]


def _stage_b_body(zc_ref, cp_ref, s1_ref, wl_ref, wr_ref, b_ref,
                  y2_ref, s2_ref):
  z = jnp.concatenate((zc_ref[0], zc_ref[1]), axis=-1).astype(jnp.float32)
  cnt = jnp.max(cp_ref[0] + cp_ref[1], axis=1)
  inv = 1.0 / jnp.maximum(cnt, 1.0)
  h = jnp.maximum(z * inv[:, None] + s1_ref[...], 0.0)
  y2 = _dot(h, wl_ref[...]).astype(jnp.bfloat16)
  y2_ref[0] = y2[:, :_KW]
  y2_ref[1] = y2[:, _KW:]
  s2_ref[...] = _dot(h, wr_ref[...]) + b_ref[...]


def _stage_c_body(zc_ref, cp_ref, s2_ref, out_ref):
  z = jnp.concatenate((zc_ref[0], zc_ref[1]), axis=-1).astype(jnp.float32)
  cnt = jnp.max(cp_ref[0] + cp_ref[1], axis=1)
  inv = 1.0 / jnp.maximum(cnt, 1.0)
  out_ref[...] = z * inv[:, None] + s2_ref[...]


_row_spec = pl.BlockSpec((_BLK, _D), lambda i: (i, 0))
_w_spec = pl.BlockSpec((_D, _D), lambda i: (0, 0))
_b_spec = pl.BlockSpec((1, _D), lambda i: (0, 0))
_ys_spec = pl.BlockSpec((_NC, _BLK, _KW), lambda i: (0, i, 0))
_zc_spec = pl.BlockSpec((_NC, _BLK, _KW), lambda i: (0, i, 0))
_cp_spec = pl.BlockSpec((_NC, _BLK, 16), lambda i: (0, i, 0))

_ys_shape = jax.ShapeDtypeStruct((_NC, _N, _KW), jnp.bfloat16)
_s_shape = jax.ShapeDtypeStruct((_N, _D), jnp.float32)

_stage_a = pl.pallas_call(
    _stage_a_body, grid=(_G,),
    in_specs=[_row_spec, _w_spec, _w_spec, _b_spec],
    out_specs=[_ys_spec, _row_spec],
    out_shape=[_ys_shape, _s_shape],
)

_stage_b = pl.pallas_call(
    _stage_b_body, grid=(_G,),
    in_specs=[_zc_spec, _cp_spec, _row_spec, _w_spec, _w_spec, _b_spec],
    out_specs=[_ys_spec, _row_spec],
    out_shape=[_ys_shape, _s_shape],
)

_stage_c = pl.pallas_call(
    _stage_c_body, grid=(_G,),
    in_specs=[_zc_spec, _cp_spec, _row_spec],
    out_specs=_row_spec,
    out_shape=_s_shape,
)


def kernel(x, edge_index, W1l, b1l, W1r, W2l, b2l, W2r):
  src = edge_index[0]
  dst = edge_index[1]
  src_p = jnp.zeros((_EPAD,), jnp.int32).at[:_E].set(src)
  dstr = jnp.full((_EPAD,), _N, jnp.int32).at[:_E].set(dst)
  dstr = dstr.reshape(_NS, _CHT, _K)
  srcr = jnp.stack([src_p, src_p + _N]).reshape(_NC, _NS, _CHT, _K)

  b1 = b1l.reshape(1, _D)
  b2 = b2l.reshape(1, _D)

  y1, s1 = _stage_a(x, W1l, W1r, b1)
  z1, cp = _prop_cnt(y1.reshape(_NC * _N, _KW), srcr, dstr)
  y2, s2 = _stage_b(z1, cp, s1, W2l, W2r, b2)
  z2, _unused_cp2 = _prop_cnt(y2.reshape(_NC * _N, _KW), srcr, dstr)
  return _stage_c(z2, cp, s2)


# speedup vs baseline: 1.8042x; 1.8042x over previous
import jax
import jax.numpy as jnp
from jax import lax
from jax.experimental import pallas as pl
from jax.experimental.pallas import tpu as pltpu
from jax.experimental.pallas import tpu_sc as plsc

_N = 10000
_E = 320000
_D = 128
_KW = 64
_NC = 2
_NS = 16
_K = 256
_CHT = 80
_EPAD = _NS * _CHT * _K
_NPAD = 10240
_RPT = _NPAD // _NS
_NBUF = 4

_HI = lax.Precision.HIGHEST


def _make_propagate(with_cnt: bool):
  mesh = plsc.VectorSubcoreMesh(core_axis_name="c", subcore_axis_name="s")
  out_type = [jax.ShapeDtypeStruct((_NC, _NPAD, _KW), jnp.bfloat16)]
  if with_cnt:
    out_type.append(jax.ShapeDtypeStruct((_NC, _NPAD, 16), jnp.float32))
  scratch = [
      pltpu.VMEM((_CHT, _K), jnp.int32),
      pltpu.VMEM((_CHT, _K), jnp.int32),
  ]
  scratch += [pltpu.VMEM((_K, _KW), jnp.bfloat16) for _ in range(_NBUF)]
  if with_cnt:
    scratch.append(pltpu.VMEM((_K, 16), jnp.float32))
    scratch.append(pltpu.VMEM_SHARED((_NPAD, 16), jnp.float32))
  scratch.append(pltpu.VMEM_SHARED((_NPAD, _KW), jnp.bfloat16))
  scratch += [pltpu.SemaphoreType.DMA for _ in range(2 * _NBUF)]

  def body(y_hbm, src_hbm, dst_hbm, *refs):
    if with_cnt:
      z_out, cnt_out = refs[0], refs[1]
      rest = refs[2:]
    else:
      z_out = refs[0]
      rest = refs[1:]
    idx_s, idx_d = rest[0], rest[1]
    bufs = rest[2:2 + _NBUF]
    off = 2 + _NBUF
    if with_cnt:
      ones_b, cnt_sh = rest[off], rest[off + 1]
      off += 2
    z_sh = rest[off]
    sems = rest[off + 1:]
    sem_g, sem_s = sems[:_NBUF], sems[_NBUF:]

    cid = lax.axis_index("c")
    sid = lax.axis_index("s")
    base = sid * _RPT

    pltpu.sync_copy(src_hbm.at[cid, sid], idx_s)
    pltpu.sync_copy(dst_hbm.at[sid], idx_d)

    zv = jnp.zeros((16,), jnp.float32)
    zvh = jnp.zeros((32,), jnp.bfloat16)

    def _zrow(i, c):
      def _zcol(j, c2):
        bufs[0][i, pl.ds(j * 32, 32)] = zvh
        return c2
      return lax.fori_loop(0, _KW // 32, _zcol, c)
    lax.fori_loop(0, _K, _zrow, 0)

    nfull, rem = divmod(_RPT, _K)
    for r in range(nfull):
      pltpu.sync_copy(bufs[0], z_sh.at[pl.ds(base + r * _K, _K)])
    if rem:
      pltpu.sync_copy(bufs[0].at[pl.ds(0, rem)],
                      z_sh.at[pl.ds(base + nfull * _K, rem)])

    if with_cnt:
      ov = jnp.ones((16,), jnp.float32)

      def _zofill(i, c):
        ones_b[i, :] = zv
        return c
      lax.fori_loop(0, _K, _zofill, 0)
      for r in range(nfull):
        pltpu.sync_copy(ones_b, cnt_sh.at[pl.ds(base + r * _K, _K)])
      if rem:
        pltpu.sync_copy(ones_b.at[pl.ds(0, rem)],
                        cnt_sh.at[pl.ds(base + nfull * _K, rem)])

      def _onefill(i, c):
        ones_b[i, :] = ov
        return c
      lax.fori_loop(0, _K, _onefill, 0)

    plsc.subcore_barrier()

    def _cnt_pred(j):
      return (j >= cid * (_CHT // 2)) & (j < (cid + 1) * (_CHT // 2))

    def _cnt_chunk(j, p):
      @pl.when(_cnt_pred(j))
      def _():
        pltpu.async_copy(ones_b, cnt_sh.at[idx_d.at[j]], sem_s[p], add=True)

    def _wait_cnt_chunk(j, p):
      @pl.when(_cnt_pred(j))
      def _():
        pltpu.make_async_copy(ones_b, cnt_sh.at[idx_d.at[j]], sem_s[p]).wait()

    def _gather(j, p):
      pltpu.async_copy(y_hbm.at[idx_s.at[j]], bufs[p], sem_g[p])

    def _wait_gather(j, p):
      pltpu.make_async_copy(y_hbm.at[idx_s.at[j]], bufs[p], sem_g[p]).wait()

    def _scatter(j, p):
      pltpu.async_copy(bufs[p], z_sh.at[idx_d.at[j]], sem_s[p], add=True)

    def _wait_scatter(j, p):
      pltpu.make_async_copy(bufs[p], z_sh.at[idx_d.at[j]], sem_s[p]).wait()

    for p in range(_NBUF):
      _gather(p, p)

    def _step(t, c):
      for p in range(_NBUF):
        j = _NBUF * t + p
        _wait_gather(j, p)
        _scatter(j, p)
        if with_cnt:
          _cnt_chunk(j, p)
        _wait_scatter(j, p)
        if with_cnt:
          _wait_cnt_chunk(j, p)
        _gather(j + _NBUF, p)
      return c
    lax.fori_loop(0, _CHT // _NBUF - 1, _step, 0)

    for p in range(_NBUF):
      j = _CHT - _NBUF + p
      _wait_gather(j, p)
      _scatter(j, p)
      if with_cnt:
        _cnt_chunk(j, p)
      _wait_scatter(j, p)
      if with_cnt:
        _wait_cnt_chunk(j, p)

    plsc.subcore_barrier()
    pltpu.sync_copy(z_sh.at[pl.ds(base, _RPT)],
                    z_out.at[cid, pl.ds(base, _RPT)])
    if with_cnt:
      pltpu.sync_copy(cnt_sh.at[pl.ds(base, _RPT)],
                      cnt_out.at[cid, pl.ds(base, _RPT)])

  return pl.kernel(body, out_type=tuple(out_type), mesh=mesh,
                   scratch_types=tuple(scratch),
                   compiler_params=pltpu.CompilerParams(
                       use_tc_tiling_on_sc=False))


_prop_cnt = _make_propagate(True)
_prop = _make_propagate(False)


_BLK = 1000
_G = _N // _BLK


def _dot(a, b):
  return lax.dot_general(a, b, (((1,), (0,)), ((), ())), precision=_HI)


def _stage_a_body(x_ref, wl_ref, wr_ref, b_ref, y_ref, s_ref):
  xb = x_ref[...]
  y = _dot(xb, wl_ref[...]).astype(jnp.bfloat16)
  y_ref[0] = y[:, :_KW]
  y_ref[1] = y[:, _KW:]
  s_ref[...] = _dot(xb, wr_ref[...]) + b_ref[...]


def _stage_b_body(zc_ref, cp_ref, s1_ref, wl_ref, wr_ref, b_ref,
                  y2_ref, s2_ref):
  z = jnp.concatenate((zc_ref[0], zc_ref[1]), axis=-1).astype(jnp.float32)
  cnt = jnp.max(cp_ref[0] + cp_ref[1], axis=1)
  inv = 1.0 / jnp.maximum(cnt, 1.0)
  h = jnp.maximum(z * inv[:, None] + s1_ref[...], 0.0)
  y2 = _dot(h, wl_ref[...]).astype(jnp.bfloat16)
  y2_ref[0] = y2[:, :_KW]
  y2_ref[1] = y2[:, _KW:]
  s2_ref[...] = _dot(h, wr_ref[...]) + b_ref[...]


def _stage_c_body(zc_ref, cp_ref, s2_ref, out_ref):
  z = jnp.concatenate((zc_ref[0], zc_ref[1]), axis=-1).astype(jnp.float32)
  cnt = jnp.max(cp_ref[0] + cp_ref[1], axis=1)
  inv = 1.0 / jnp.maximum(cnt, 1.0)
  out_ref[...] = z * inv[:, None] + s2_ref[...]


_row_spec = pl.BlockSpec((_BLK, _D), lambda i: (i, 0))
_w_spec = pl.BlockSpec((_D, _D), lambda i: (0, 0))
_b_spec = pl.BlockSpec((1, _D), lambda i: (0, 0))
_ys_spec = pl.BlockSpec((_NC, _BLK, _KW), lambda i: (0, i, 0))
_zc_spec = pl.BlockSpec((_NC, _BLK, _KW), lambda i: (0, i, 0))
_cp_spec = pl.BlockSpec((_NC, _BLK, 16), lambda i: (0, i, 0))

_ys_shape = jax.ShapeDtypeStruct((_NC, _N, _KW), jnp.bfloat16)
_s_shape = jax.ShapeDtypeStruct((_N, _D), jnp.float32)

_stage_a = pl.pallas_call(
    _stage_a_body, grid=(_G,),
    in_specs=[_row_spec, _w_spec, _w_spec, _b_spec],
    out_specs=[_ys_spec, _row_spec],
    out_shape=[_ys_shape, _s_shape],
)

_stage_b = pl.pallas_call(
    _stage_b_body, grid=(_G,),
    in_specs=[_zc_spec, _cp_spec, _row_spec, _w_spec, _w_spec, _b_spec],
    out_specs=[_ys_spec, _row_spec],
    out_shape=[_ys_shape, _s_shape],
)

_stage_c = pl.pallas_call(
    _stage_c_body, grid=(_G,),
    in_specs=[_zc_spec, _cp_spec, _row_spec],
    out_specs=_row_spec,
    out_shape=_s_shape,
)


def kernel(x, edge_index, W1l, b1l, W1r, W2l, b2l, W2r):
  src = edge_index[0]
  dst = edge_index[1]
  fill = jnp.arange(_EPAD, dtype=jnp.int32)
  src_p = (fill % _N).at[:_E].set(src)
  dstr = (_N + fill % (_NPAD - _N)).at[:_E].set(dst)
  dstr = dstr.reshape(_NS, _CHT, _K)
  srcr = jnp.stack([src_p, src_p + _N]).reshape(_NC, _NS, _CHT, _K)

  b1 = b1l.reshape(1, _D)
  b2 = b2l.reshape(1, _D)

  y1, s1 = _stage_a(x, W1l, W1r, b1)
  z1, cp = _prop_cnt(y1.reshape(_NC * _N, _KW), srcr, dstr)
  y2, s2 = _stage_b(z1, cp, s1, W2l, W2r, b2)
  (z2,) = _prop(y2.reshape(_NC * _N, _KW), srcr, dstr)
  return _stage_c(z2, cp, s2)


# speedup vs baseline: 1.8718x; 1.0375x over previous
import jax
import jax.numpy as jnp
from jax import lax
from jax.experimental import pallas as pl
from jax.experimental.pallas import tpu as pltpu
from jax.experimental.pallas import tpu_sc as plsc

_N = 10000
_E = 320000
_D = 128
_KW = 64
_NC = 2
_NS = 16
_K = 256
_CHT = 80
_EPAD = _NS * _CHT * _K
_NPAD = 10240
_RPT = _NPAD // _NS
_NBUF = 4

_HI = lax.Precision.HIGHEST


def _make_propagate(with_cnt: bool):
  mesh = plsc.VectorSubcoreMesh(core_axis_name="c", subcore_axis_name="s")
  out_type = [jax.ShapeDtypeStruct((_NC, _NPAD, _KW), jnp.bfloat16)]
  if with_cnt:
    out_type.append(jax.ShapeDtypeStruct((_NC, _NPAD, 16), jnp.float32))
  scratch = [
      pltpu.VMEM((_CHT, _K), jnp.int32),
      pltpu.VMEM((_CHT, _K), jnp.int32),
  ]
  scratch += [pltpu.VMEM((_K, _KW), jnp.bfloat16) for _ in range(_NBUF)]
  if with_cnt:
    scratch.append(pltpu.VMEM((_K, 16), jnp.float32))
    scratch.append(pltpu.VMEM_SHARED((_NPAD, 16), jnp.float32))
  scratch.append(pltpu.VMEM_SHARED((_NPAD, _KW), jnp.bfloat16))
  scratch += [pltpu.SemaphoreType.DMA for _ in range(2 * _NBUF)]

  def body(y_hbm, src_hbm, dst_hbm, *refs):
    if with_cnt:
      z_out, cnt_out = refs[0], refs[1]
      rest = refs[2:]
    else:
      z_out = refs[0]
      rest = refs[1:]
    idx_s, idx_d = rest[0], rest[1]
    bufs = rest[2:2 + _NBUF]
    off = 2 + _NBUF
    if with_cnt:
      ones_b, cnt_sh = rest[off], rest[off + 1]
      off += 2
    z_sh = rest[off]
    sems = rest[off + 1:]
    sem_g, sem_s = sems[:_NBUF], sems[_NBUF:]

    cid = lax.axis_index("c")
    sid = lax.axis_index("s")
    base = sid * _RPT

    pltpu.sync_copy(src_hbm.at[cid, sid], idx_s)
    pltpu.sync_copy(dst_hbm.at[sid], idx_d)

    zv = jnp.zeros((16,), jnp.float32)
    zvh = jnp.zeros((32,), jnp.bfloat16)

    def _zrow(i, c):
      def _zcol(j, c2):
        bufs[0][i, pl.ds(j * 32, 32)] = zvh
        return c2
      return lax.fori_loop(0, _KW // 32, _zcol, c)
    lax.fori_loop(0, _K, _zrow, 0)

    nfull, rem = divmod(_RPT, _K)
    for r in range(nfull):
      pltpu.sync_copy(bufs[0], z_sh.at[pl.ds(base + r * _K, _K)])
    if rem:
      pltpu.sync_copy(bufs[0].at[pl.ds(0, rem)],
                      z_sh.at[pl.ds(base + nfull * _K, rem)])

    if with_cnt:
      ov = jnp.ones((16,), jnp.float32)

      def _zofill(i, c):
        ones_b[i, :] = zv
        return c
      lax.fori_loop(0, _K, _zofill, 0)
      for r in range(nfull):
        pltpu.sync_copy(ones_b, cnt_sh.at[pl.ds(base + r * _K, _K)])
      if rem:
        pltpu.sync_copy(ones_b.at[pl.ds(0, rem)],
                        cnt_sh.at[pl.ds(base + nfull * _K, rem)])

      def _onefill(i, c):
        ones_b[i, :] = ov
        return c
      lax.fori_loop(0, _K, _onefill, 0)

    plsc.subcore_barrier()

    def _cnt_pred(j):
      return (j >= cid * (_CHT // 2)) & (j < (cid + 1) * (_CHT // 2))

    def _cnt_chunk(j, p):
      @pl.when(_cnt_pred(j))
      def _():
        pltpu.async_copy(ones_b, cnt_sh.at[idx_d.at[j]], sem_s[p], add=True)

    def _wait_cnt_chunk(j, p):
      @pl.when(_cnt_pred(j))
      def _():
        pltpu.make_async_copy(ones_b, cnt_sh.at[idx_d.at[j]], sem_s[p]).wait()

    def _gather(j, p):
      pltpu.async_copy(y_hbm.at[idx_s.at[j]], bufs[p], sem_g[p])

    def _wait_gather(j, p):
      pltpu.make_async_copy(y_hbm.at[idx_s.at[j]], bufs[p], sem_g[p]).wait()

    def _scatter(j, p):
      pltpu.async_copy(bufs[p], z_sh.at[idx_d.at[j]], sem_s[p], add=True)

    def _wait_scatter(j, p):
      pltpu.make_async_copy(bufs[p], z_sh.at[idx_d.at[j]], sem_s[p]).wait()

    for p in range(_NBUF):
      _gather(p, p)

    def _step(t, c):
      for p in range(_NBUF):
        j = _NBUF * t + p
        _wait_gather(j, p)
        _scatter(j, p)
        if with_cnt:
          _cnt_chunk(j, p)
        _wait_scatter(j, p)
        if with_cnt:
          _wait_cnt_chunk(j, p)
        _gather(j + _NBUF, p)
      return c
    lax.fori_loop(0, _CHT // _NBUF - 1, _step, 0)

    for p in range(_NBUF):
      j = _CHT - _NBUF + p
      _wait_gather(j, p)
      _scatter(j, p)
      if with_cnt:
        _cnt_chunk(j, p)
      _wait_scatter(j, p)
      if with_cnt:
        _wait_cnt_chunk(j, p)

    plsc.subcore_barrier()
    pltpu.sync_copy(z_sh.at[pl.ds(base, _RPT)],
                    z_out.at[cid, pl.ds(base, _RPT)])
    if with_cnt:
      pltpu.sync_copy(cnt_sh.at[pl.ds(base, _RPT)],
                      cnt_out.at[cid, pl.ds(base, _RPT)])

  return pl.kernel(body, out_type=tuple(out_type), mesh=mesh,
                   scratch_types=tuple(scratch),
                   compiler_params=pltpu.CompilerParams(
                       use_tc_tiling_on_sc=False))


_prop_cnt = _make_propagate(True)
_prop = _make_propagate(False)


_BLK = 1000
_G = _N // _BLK


def _dot(a, b):
  return lax.dot_general(a, b, (((1,), (0,)), ((), ())), precision=_HI)


def _stage_a1_body(x_ref, wl_ref, y_ref):
  y = _dot(x_ref[...], wl_ref[...]).astype(jnp.bfloat16)
  y_ref[0] = y[:, :_KW]
  y_ref[1] = y[:, _KW:]


def _stage_a2_body(x_ref, wr_ref, b_ref, s_ref):
  s_ref[...] = _dot(x_ref[...], wr_ref[...]) + b_ref[...]


def _relu_h(zc_ref, cp_ref, s1_ref):
  z = jnp.concatenate((zc_ref[0], zc_ref[1]), axis=-1).astype(jnp.float32)
  cnt = jnp.max(cp_ref[0] + cp_ref[1], axis=1)
  inv = 1.0 / jnp.maximum(cnt, 1.0)
  return jnp.maximum(z * inv[:, None] + s1_ref[...], 0.0)


def _stage_b1_body(zc_ref, cp_ref, s1_ref, wl_ref, y2_ref):
  y2 = _dot(_relu_h(zc_ref, cp_ref, s1_ref), wl_ref[...]).astype(jnp.bfloat16)
  y2_ref[0] = y2[:, :_KW]
  y2_ref[1] = y2[:, _KW:]


def _stage_b2_body(zc_ref, cp_ref, s1_ref, wr_ref, b_ref, s2_ref):
  s2_ref[...] = _dot(_relu_h(zc_ref, cp_ref, s1_ref), wr_ref[...]) + b_ref[...]


def _stage_c_body(zc_ref, cp_ref, s2_ref, out_ref):
  z = jnp.concatenate((zc_ref[0], zc_ref[1]), axis=-1).astype(jnp.float32)
  cnt = jnp.max(cp_ref[0] + cp_ref[1], axis=1)
  inv = 1.0 / jnp.maximum(cnt, 1.0)
  out_ref[...] = z * inv[:, None] + s2_ref[...]


_row_spec = pl.BlockSpec((_BLK, _D), lambda i: (i, 0))
_w_spec = pl.BlockSpec((_D, _D), lambda i: (0, 0))
_b_spec = pl.BlockSpec((1, _D), lambda i: (0, 0))
_ys_spec = pl.BlockSpec((_NC, _BLK, _KW), lambda i: (0, i, 0))
_zc_spec = pl.BlockSpec((_NC, _BLK, _KW), lambda i: (0, i, 0))
_cp_spec = pl.BlockSpec((_NC, _BLK, 16), lambda i: (0, i, 0))

_ys_shape = jax.ShapeDtypeStruct((_NC, _N, _KW), jnp.bfloat16)
_s_shape = jax.ShapeDtypeStruct((_N, _D), jnp.float32)

_stage_a1 = pl.pallas_call(
    _stage_a1_body, grid=(_G,),
    in_specs=[_row_spec, _w_spec],
    out_specs=_ys_spec,
    out_shape=_ys_shape,
)

_stage_a2 = pl.pallas_call(
    _stage_a2_body, grid=(_G,),
    in_specs=[_row_spec, _w_spec, _b_spec],
    out_specs=_row_spec,
    out_shape=_s_shape,
)

_stage_b1 = pl.pallas_call(
    _stage_b1_body, grid=(_G,),
    in_specs=[_zc_spec, _cp_spec, _row_spec, _w_spec],
    out_specs=_ys_spec,
    out_shape=_ys_shape,
)

_stage_b2 = pl.pallas_call(
    _stage_b2_body, grid=(_G,),
    in_specs=[_zc_spec, _cp_spec, _row_spec, _w_spec, _b_spec],
    out_specs=_row_spec,
    out_shape=_s_shape,
)

_stage_c = pl.pallas_call(
    _stage_c_body, grid=(_G,),
    in_specs=[_zc_spec, _cp_spec, _row_spec],
    out_specs=_row_spec,
    out_shape=_s_shape,
)


def kernel(x, edge_index, W1l, b1l, W1r, W2l, b2l, W2r):
  src = edge_index[0]
  dst = edge_index[1]
  fill = jnp.arange(_EPAD, dtype=jnp.int32)
  src_p = (fill % _N).at[:_E].set(src)
  dstr = (_N + fill % (_NPAD - _N)).at[:_E].set(dst)
  dstr = dstr.reshape(_NS, _CHT, _K)
  srcr = jnp.stack([src_p, src_p + _N]).reshape(_NC, _NS, _CHT, _K)

  b1 = b1l.reshape(1, _D)
  b2 = b2l.reshape(1, _D)

  y1 = _stage_a1(x, W1l)
  z1, cp = _prop_cnt(y1.reshape(_NC * _N, _KW), srcr, dstr)
  s1 = _stage_a2(x, W1r, b1)
  y2 = _stage_b1(z1, cp, s1, W2l)
  (z2,) = _prop(y2.reshape(_NC * _N, _KW), srcr, dstr)
  s2 = _stage_b2(z1, cp, s1, W2r, b2)
  return _stage_c(z2, cp, s2)
